# baseline (speedup 1.0000x reference)
import jax
import jax.numpy as jnp
from jax.experimental import pallas as pl
from jax.experimental.pallas import tpu as pltpu

_COST_CLASS = 1.0
_COST_BBOX = 5.0
_COST_GIOU = 2.0

_TQ = 36


def _cost_kernel(lb_ref, labels_ref, tbt_ref, out_ref):
    tq, b_dim, c4_dim = lb_ref.shape
    c_dim = c4_dim - 4
    rows = tq * b_dim
    lb = lb_ref[...].reshape(rows, c4_dim)
    logits = lb[:, :c_dim]
    mx = jnp.max(logits, axis=-1, keepdims=True)
    e = jnp.exp(logits - mx)
    prob = e / jnp.sum(e, axis=-1, keepdims=True)

    labels = labels_ref[...]
    t_dim = labels.shape[-1]
    iota_c = jax.lax.broadcasted_iota(jnp.int32, (c_dim, t_dim), 0)
    sel = jnp.where(iota_c == labels, jnp.float32(_COST_GIOU - _COST_CLASS),
                    jnp.float32(_COST_GIOU))
    two_minus_prob_at = jnp.dot(prob, sel, preferred_element_type=jnp.float32)

    qb = lb[:, c_dim:].astype(jnp.bfloat16)
    qcx, qcy = qb[:, 0:1], qb[:, 1:2]
    qw, qh = qb[:, 2:3], qb[:, 3:4]
    tbt16 = tbt_ref[...].astype(jnp.bfloat16)
    tcx, tcy = tbt16[0:1, :], tbt16[1:2, :]
    tw, th = tbt16[2:3, :], tbt16[3:4, :]

    s1 = jnp.abs(qcx - tcx) + jnp.abs(qcy - tcy)
    s2 = jnp.abs(qw - tw) + jnp.abs(qh - th)
    cost_bbox = s1.astype(jnp.float32) + s2.astype(jnp.float32)

    qx1, qx2 = qcx - 0.5 * qw, qcx + 0.5 * qw
    qy1, qy2 = qcy - 0.5 * qh, qcy + 0.5 * qh
    tx1, tx2 = tcx - 0.5 * tw, tcx + 0.5 * tw
    ty1, ty2 = tcy - 0.5 * th, tcy + 0.5 * th

    zero = jnp.bfloat16(0.0)
    dx = jnp.minimum(qx2, tx2) - jnp.maximum(qx1, tx1)
    dy = jnp.minimum(qy2, ty2) - jnp.maximum(qy1, ty1)
    inter = jnp.maximum(dx, zero) * jnp.maximum(dy, zero)
    area_q = qw * qh
    area_t = tw * th
    union = (area_q + area_t) - inter
    area_e = ((qw + tw) - dx) * ((qh + th) - dy)
    giou_sum = inter / union + union / area_e
    cost = ((_COST_BBOX * cost_bbox + two_minus_prob_at)
            - _COST_GIOU * giou_sum.astype(jnp.float32))
    out_ref[...] = cost.reshape(tq, b_dim, t_dim)


def _cost_call(lb_t, labels2d, tbt):
    q_dim, b_dim, c4_dim = lb_t.shape
    t_dim = labels2d.shape[1]
    q_tiles = (q_dim + _TQ - 1) // _TQ
    return pl.pallas_call(
        _cost_kernel,
        out_shape=jax.ShapeDtypeStruct((q_dim, b_dim, t_dim), jnp.float32),
        grid=(q_tiles,),
        in_specs=[
            pl.BlockSpec((_TQ, b_dim, c4_dim), lambda q: (q, 0, 0)),
            pl.BlockSpec((1, t_dim), lambda q: (0, 0)),
            pl.BlockSpec((4, t_dim), lambda q: (0, 0)),
        ],
        out_specs=pl.BlockSpec((_TQ, b_dim, t_dim), lambda q: (q, 0, 0)),
        compiler_params=pltpu.CompilerParams(
            dimension_semantics=("parallel",),
            vmem_limit_bytes=56 * 1024 * 1024,
        ),
        name="hungarian_cost",
    )(lb_t, labels2d, tbt)


def kernel(pred_logits, pred_boxes, tgt_labels, tgt_boxes):
    t_dim = tgt_labels.shape[0]
    labels2d = tgt_labels.astype(jnp.int32).reshape(1, t_dim)
    tbt = tgt_boxes.T
    lb = jnp.concatenate([pred_logits, pred_boxes], axis=2)
    lb_t = jnp.transpose(lb, (1, 0, 2))
    out_t = _cost_call(lb_t, labels2d, tbt)
    return jnp.transpose(out_t, (1, 0, 2))


# speedup vs baseline: 1.0791x; 1.0791x over previous
import jax
import jax.numpy as jnp
from jax.experimental import pallas as pl
from jax.experimental.pallas import tpu as pltpu

_COST_CLASS = 1.0
_COST_BBOX = 5.0
_COST_GIOU = 2.0

_TQ = 36


def _cost_kernel(logits_ref, boxes_ref, labels_ref, tbt_ref, out_ref):
    tq, b_dim, c_dim = logits_ref.shape
    rows = tq * b_dim
    logits = logits_ref[...].reshape(rows, c_dim)
    mx = jnp.max(logits, axis=-1, keepdims=True)
    e = jnp.exp(logits - mx)
    prob = e / jnp.sum(e, axis=-1, keepdims=True)

    labels = labels_ref[...]
    t_dim = labels.shape[-1]
    iota_c = jax.lax.broadcasted_iota(jnp.int32, (c_dim, t_dim), 0)
    sel = jnp.where(iota_c == labels, jnp.float32(_COST_GIOU - _COST_CLASS),
                    jnp.float32(_COST_GIOU))
    two_minus_prob_at = jnp.dot(prob, sel, preferred_element_type=jnp.float32)

    qb = boxes_ref[...].reshape(rows, 4).astype(jnp.bfloat16)
    qcx, qcy = qb[:, 0:1], qb[:, 1:2]
    qw, qh = qb[:, 2:3], qb[:, 3:4]
    tbt16 = tbt_ref[...].astype(jnp.bfloat16)
    tcx, tcy = tbt16[0:1, :], tbt16[1:2, :]
    tw, th = tbt16[2:3, :], tbt16[3:4, :]

    s1 = jnp.abs(qcx - tcx) + jnp.abs(qcy - tcy)
    s2 = jnp.abs(qw - tw) + jnp.abs(qh - th)
    cost_bbox = s1.astype(jnp.float32) + s2.astype(jnp.float32)

    qx1, qx2 = qcx - 0.5 * qw, qcx + 0.5 * qw
    qy1, qy2 = qcy - 0.5 * qh, qcy + 0.5 * qh
    tx1, tx2 = tcx - 0.5 * tw, tcx + 0.5 * tw
    ty1, ty2 = tcy - 0.5 * th, tcy + 0.5 * th

    zero = jnp.bfloat16(0.0)
    dx = jnp.minimum(qx2, tx2) - jnp.maximum(qx1, tx1)
    dy = jnp.minimum(qy2, ty2) - jnp.maximum(qy1, ty1)
    inter = jnp.maximum(dx, zero) * jnp.maximum(dy, zero)
    area_q = qw * qh
    area_t = tw * th
    union = (area_q + area_t) - inter
    area_e = ((qw + tw) - dx) * ((qh + th) - dy)
    giou_sum = inter / union + union / area_e
    cost = ((_COST_BBOX * cost_bbox + two_minus_prob_at)
            - _COST_GIOU * giou_sum.astype(jnp.float32))
    out_ref[...] = cost.reshape(tq, b_dim, t_dim)


def _cost_call(logits_t, boxes_t, labels2d, tbt):
    q_dim, b_dim, c_dim = logits_t.shape
    t_dim = labels2d.shape[1]
    q_tiles = (q_dim + _TQ - 1) // _TQ
    return pl.pallas_call(
        _cost_kernel,
        out_shape=jax.ShapeDtypeStruct((q_dim, b_dim, t_dim), jnp.float32),
        grid=(q_tiles,),
        in_specs=[
            pl.BlockSpec((_TQ, b_dim, c_dim), lambda q: (q, 0, 0)),
            pl.BlockSpec((_TQ, b_dim, 4), lambda q: (q, 0, 0)),
            pl.BlockSpec((1, t_dim), lambda q: (0, 0)),
            pl.BlockSpec((4, t_dim), lambda q: (0, 0)),
        ],
        out_specs=pl.BlockSpec((_TQ, b_dim, t_dim), lambda q: (q, 0, 0)),
        compiler_params=pltpu.CompilerParams(
            dimension_semantics=("parallel",),
            vmem_limit_bytes=56 * 1024 * 1024,
        ),
        name="hungarian_cost",
    )(logits_t, boxes_t, labels2d, tbt)


def kernel(pred_logits, pred_boxes, tgt_labels, tgt_boxes):
    t_dim = tgt_labels.shape[0]
    labels2d = tgt_labels.astype(jnp.int32).reshape(1, t_dim)
    tbt = tgt_boxes.T
    logits_t = jnp.transpose(pred_logits, (1, 0, 2))
    boxes_t = jnp.transpose(pred_boxes, (1, 0, 2))
    out_t = _cost_call(logits_t, boxes_t, labels2d, tbt)
    return jnp.transpose(out_t, (1, 0, 2))


# speedup vs baseline: 1.0905x; 1.0106x over previous
import jax
import jax.numpy as jnp
from jax.experimental import pallas as pl
from jax.experimental.pallas import tpu as pltpu

_COST_CLASS = 1.0
_COST_BBOX = 5.0
_COST_GIOU = 2.0

_TQ = 36


def _cost_kernel(logits_ref, boxes_ref, labels_ref, tbt_ref, out_ref):
    tq, b_dim, c_dim = logits_ref.shape
    rows = tq * b_dim
    logits = logits_ref[...].reshape(rows, c_dim)
    mx = jnp.max(logits, axis=-1, keepdims=True)
    e = jnp.exp(logits - mx)
    prob = e / jnp.sum(e, axis=-1, keepdims=True)

    labels = labels_ref[...].reshape(1, -1)
    t_dim = labels.shape[-1]
    iota_c = jax.lax.broadcasted_iota(jnp.int32, (c_dim, t_dim), 0)
    sel = jnp.where(iota_c == labels, jnp.float32(_COST_GIOU - _COST_CLASS),
                    jnp.float32(_COST_GIOU))
    two_minus_prob_at = jnp.dot(prob, sel, preferred_element_type=jnp.float32)

    qb = boxes_ref[...].reshape(rows, 4).astype(jnp.bfloat16)
    qcx, qcy = qb[:, 0:1], qb[:, 1:2]
    qw, qh = qb[:, 2:3], qb[:, 3:4]
    tbt16 = tbt_ref[...].astype(jnp.bfloat16)
    tcx, tcy = tbt16[0:1, :], tbt16[1:2, :]
    tw, th = tbt16[2:3, :], tbt16[3:4, :]

    s1 = jnp.abs(qcx - tcx) + jnp.abs(qcy - tcy)
    s2 = jnp.abs(qw - tw) + jnp.abs(qh - th)
    cost_bbox = s1.astype(jnp.float32) + s2.astype(jnp.float32)

    qx1, qx2 = qcx - 0.5 * qw, qcx + 0.5 * qw
    qy1, qy2 = qcy - 0.5 * qh, qcy + 0.5 * qh
    tx1, tx2 = tcx - 0.5 * tw, tcx + 0.5 * tw
    ty1, ty2 = tcy - 0.5 * th, tcy + 0.5 * th

    zero = jnp.bfloat16(0.0)
    dx = jnp.minimum(qx2, tx2) - jnp.maximum(qx1, tx1)
    dy = jnp.minimum(qy2, ty2) - jnp.maximum(qy1, ty1)
    inter = jnp.maximum(dx, zero) * jnp.maximum(dy, zero)
    area_q = qw * qh
    area_t = tw * th
    union = (area_q + area_t) - inter
    area_e = ((qw + tw) - dx) * ((qh + th) - dy)
    giou_sum = inter / union + union / area_e
    cost = ((_COST_BBOX * cost_bbox + two_minus_prob_at)
            - _COST_GIOU * giou_sum.astype(jnp.float32))
    out_ref[...] = cost.reshape(tq, b_dim, t_dim)


def _cost_call(logits_t, boxes_t, labels1d, tbt):
    q_dim, b_dim, c_dim = logits_t.shape
    t_dim = labels1d.shape[0]
    q_tiles = (q_dim + _TQ - 1) // _TQ
    return pl.pallas_call(
        _cost_kernel,
        out_shape=jax.ShapeDtypeStruct((q_dim, b_dim, t_dim), jnp.float32),
        grid=(q_tiles,),
        in_specs=[
            pl.BlockSpec((_TQ, b_dim, c_dim), lambda q: (q, 0, 0)),
            pl.BlockSpec((_TQ, b_dim, 4), lambda q: (q, 0, 0)),
            pl.BlockSpec((t_dim,), lambda q: (0,)),
            pl.BlockSpec((4, t_dim), lambda q: (0, 0)),
        ],
        out_specs=pl.BlockSpec((_TQ, b_dim, t_dim), lambda q: (q, 0, 0)),
        compiler_params=pltpu.CompilerParams(
            dimension_semantics=("parallel",),
            vmem_limit_bytes=56 * 1024 * 1024,
        ),
        name="hungarian_cost",
    )(logits_t, boxes_t, labels1d, tbt)


def kernel(pred_logits, pred_boxes, tgt_labels, tgt_boxes):
    t_dim = tgt_labels.shape[0]
    labels1d = tgt_labels.astype(jnp.int32)
    tbt = tgt_boxes.T
    logits_t = jnp.transpose(pred_logits, (1, 0, 2))
    boxes_t = jnp.transpose(pred_boxes, (1, 0, 2))
    out_t = _cost_call(logits_t, boxes_t, labels1d, tbt)
    return jnp.transpose(out_t, (1, 0, 2))
